# Initial kernel scaffold; baseline (speedup 1.0000x reference)
import jax
import jax.numpy as jnp
from jax.experimental import pallas as pl


def kernel(x, edge_index, params):
    raise NotImplementedError("write your pallas kernel here")


# speedup vs baseline: 11.5832x; 11.5832x over previous
import functools

import jax
import jax.numpy as jnp
from jax import lax
from jax.experimental import pallas as pl
from jax.experimental.pallas import tpu as pltpu
from jax.experimental.pallas import tpu_sc as plsc

F32 = jnp.float32
_BSN = 2000
_BSE = 4000


def _mlp_arrs(p):
    ls, bs = p["lins"], p["bns"]
    return [ls[0]["W"], ls[0]["b"][None, :], bs[0]["g"][None, :], bs[0]["b"][None, :],
            ls[1]["W"], ls[1]["b"][None, :], bs[1]["g"][None, :], bs[1]["b"][None, :],
            ls[2]["W"], ls[2]["b"][None, :]]


def _acc_stats(st, r0, z):
    st[r0:r0 + 1, :z.shape[1]] += jnp.sum(z, axis=0, keepdims=True)
    st[r0 + 1:r0 + 2, :z.shape[1]] += jnp.sum(z * z, axis=0, keepdims=True)


def _fin_stats(st, r0, rdst, ntot, width):
    mu = st[r0:r0 + 1, :width] * (1.0 / ntot)
    var = st[r0 + 1:r0 + 2, :width] * (1.0 / ntot) - mu * mu
    st[rdst:rdst + 1, :width] = mu
    st[rdst + 1:rdst + 2, :width] = lax.rsqrt(var + 1e-5)


def _apply_bn(st, rdst, z, g, b):
    w = z.shape[1]
    return (z - st[rdst:rdst + 1, :w]) * st[rdst + 1:rdst + 2, :w] * g + b


def _dot(a, b):
    return jnp.dot(a, b, preferred_element_type=F32)


def _full_specs(arrs):
    return [pl.BlockSpec(a.shape, lambda s, i: (0, 0)) for a in arrs]


def _node1_body(ntot, *refs):
    x_ref = refs[0]
    wa = [refs[1 + j][...] for j in range(10)]
    wb = [refs[11 + j][...] for j in range(10)]
    wc1, asv, adv = refs[21][...], refs[22][...], refs[23][...]
    src1, dst1, d2o = refs[24], refs[25], refs[26]
    sta, stb = refs[27], refs[28]
    s, i = pl.program_id(0), pl.program_id(1)

    @pl.when(jnp.logical_and(s == 0, i == 0))
    def _():
        sta[...] = jnp.zeros_like(sta)
        stb[...] = jnp.zeros_like(stb)

    x = x_ref[...]
    x3 = x[:, 0:3]
    t1a = _dot(x3, wa[0]) + wa[1]
    t1b = _dot(x3, wb[0]) + wb[1]

    @pl.when(s == 0)
    def _():
        _acc_stats(sta, 0, t1a)
        _acc_stats(stb, 0, t1b)
        src1[...] = jnp.zeros_like(src1)
        dst1[...] = jnp.zeros_like(dst1)
        d2o[...] = jnp.zeros_like(d2o)

    @pl.when(jnp.logical_and(s == 1, i == 0))
    def _():
        _fin_stats(sta, 0, 4, ntot, 64)
        _fin_stats(stb, 0, 4, ntot, 32)

    @pl.when(s >= 1)
    def _():
        a1a = jnp.maximum(_apply_bn(sta, 4, t1a, wa[2], wa[3]), 0.0)
        a1b = jnp.maximum(_apply_bn(stb, 4, t1b, wb[2], wb[3]), 0.0)
        t2a = _dot(a1a, wa[4]) + wa[5]
        t2b = _dot(a1b, wb[4]) + wb[5]

        @pl.when(s == 1)
        def _():
            _acc_stats(sta, 2, t2a)
            _acc_stats(stb, 2, t2b)
            src1[...] = jnp.zeros_like(src1)
            dst1[...] = jnp.zeros_like(dst1)
            d2o[...] = jnp.zeros_like(d2o)

        @pl.when(jnp.logical_and(s == 2, i == 0))
        def _():
            _fin_stats(sta, 2, 6, ntot, 64)
            _fin_stats(stb, 2, 6, ntot, 32)

        @pl.when(s == 2)
        def _():
            a2a = jnp.maximum(_apply_bn(sta, 6, t2a, wa[6], wa[7]), 0.0)
            a2b = jnp.maximum(_apply_bn(stb, 6, t2b, wb[6], wb[7]), 0.0)
            d1 = _dot(a2a, wa[8]) + wa[9]
            d2 = _dot(a2b, wb[8]) + wb[9]
            h0 = _dot(x, wc1)
            sa = jnp.sum(h0 * asv, axis=1, keepdims=True)
            sd = jnp.sum(h0 * adv, axis=1, keepdims=True)
            bm = x.shape[0]
            src1[...] = jnp.concatenate(
                [x3 + d1, sa, x, jnp.zeros((bm, 5), F32)], axis=1)
            dst1[...] = jnp.concatenate(
                [x3, sd, x[:, 3:4], jnp.zeros((bm, 11), F32)], axis=1)
            d2o[...] = d2


def _node1(x, p, n):
    g1 = p["conv1_1"]
    arrs = ([x] + _mlp_arrs(p["mlp1_1"]) + _mlp_arrs(p["mlp1_7"])
            + [g1["W"], g1["as"][None, :], g1["ad"][None, :]])
    nbn = n // _BSN
    in_specs = ([pl.BlockSpec((_BSN, 7), lambda s, i: (i, 0))]
                + _full_specs(arrs[1:]))
    return pl.pallas_call(
        functools.partial(_node1_body, float(n)),
        grid=(3, nbn),
        in_specs=in_specs,
        out_specs=[pl.BlockSpec((_BSN, 16), lambda s, i: (i, 0)),
                   pl.BlockSpec((_BSN, 16), lambda s, i: (i, 0)),
                   pl.BlockSpec((_BSN, 3), lambda s, i: (i, 0))],
        out_shape=[jax.ShapeDtypeStruct((n, 16), F32),
                   jax.ShapeDtypeStruct((n, 16), F32),
                   jax.ShapeDtypeStruct((n, 3), F32)],
        scratch_shapes=[pltpu.VMEM((8, 64), F32), pltpu.VMEM((8, 32), F32)],
        compiler_params=pltpu.CompilerParams(
            dimension_semantics=("arbitrary", "arbitrary")),
    )(*arrs)


def _node2_body(ntot, *refs):
    at_ref, ab_ref, x_ref, d2_ref = refs[0], refs[1], refs[2], refs[3]
    w = [refs[4 + j][...] for j in range(10)]
    g11, b11 = refs[14][...], refs[15][...]
    g12, b12 = refs[16][...], refs[17][...]
    wc1, bc1 = refs[18][...], refs[19][...]
    wc2, as2, ad2 = refs[20][...], refs[21][...], refs[22][...]
    src2, dst2 = refs[23], refs[24]
    st7, st64 = refs[25], refs[26]
    s, i = pl.program_id(0), pl.program_id(1)

    @pl.when(jnp.logical_and(s == 0, i == 0))
    def _():
        st7[...] = jnp.zeros_like(st7)
        st64[...] = jnp.zeros_like(st64)

    acc = at_ref[...] + ab_ref[...]
    out1 = _dot(acc[:, 1:8] / (acc[:, 0:1] + 1e-16), wc1) + bc1

    def wzero():
        src2[...] = jnp.zeros_like(src2)
        dst2[...] = jnp.zeros_like(dst2)

    @pl.when(s == 0)
    def _():
        _acc_stats(st7, 0, out1)
        wzero()

    @pl.when(jnp.logical_and(s == 1, i == 0))
    def _():
        _fin_stats(st7, 0, 4, ntot, 7)

    @pl.when(s >= 1)
    def _():
        z = _apply_bn(st7, 4, out1, g11, b11)
        t1 = _dot(z, w[0]) + w[1]

        @pl.when(s == 1)
        def _():
            _acc_stats(st64, 0, t1)
            wzero()

        @pl.when(jnp.logical_and(s == 2, i == 0))
        def _():
            _fin_stats(st64, 0, 4, ntot, 64)

        @pl.when(s >= 2)
        def _():
            a1 = jnp.maximum(_apply_bn(st64, 4, t1, w[2], w[3]), 0.0)
            t2 = _dot(a1, w[4]) + w[5]

            @pl.when(s == 2)
            def _():
                _acc_stats(st64, 2, t2)
                wzero()

            @pl.when(jnp.logical_and(s == 3, i == 0))
            def _():
                _fin_stats(st64, 2, 6, ntot, 64)

            @pl.when(s >= 3)
            def _():
                a2 = jnp.maximum(_apply_bn(st64, 6, t2, w[6], w[7]), 0.0)
                rh = jnp.maximum(_dot(a2, w[8]) + w[9], 0.0)

                @pl.when(s == 3)
                def _():
                    _acc_stats(st7, 2, rh)
                    wzero()

                @pl.when(jnp.logical_and(s == 4, i == 0))
                def _():
                    _fin_stats(st7, 2, 6, ntot, 7)

                @pl.when(s == 4)
                def _():
                    h = _apply_bn(st7, 6, rh, g12, b12)
                    h2 = _dot(h, wc2)
                    sa = jnp.sum(h2 * as2, axis=1, keepdims=True)
                    sd = jnp.sum(h2 * ad2, axis=1, keepdims=True)
                    x = x_ref[...]
                    bm = x.shape[0]
                    pad = jnp.zeros((bm, 5), F32)
                    src2[...] = jnp.concatenate(
                        [x[:, 0:3] + d2_ref[...], sa, h, pad], axis=1)
                    dst2[...] = jnp.concatenate(
                        [x[:, 0:3], sd, h, pad], axis=1)


def _node2(accp, x, d2, p, n):
    g1, g2 = p["conv1_1"], p["conv1_2"]
    arrs = ([accp, accp, x, d2] + _mlp_arrs(p["mlp1_2"])
            + [p["bn1_1"]["g"][None, :], p["bn1_1"]["b"][None, :],
               p["bn1_2"]["g"][None, :], p["bn1_2"]["b"][None, :],
               g1["W"], g1["b"][None, :],
               g2["W"], g2["as"][None, :], g2["ad"][None, :]])
    nbn = n // _BSN
    in_specs = ([pl.BlockSpec((_BSN, 8), lambda s, i: (i, 0)),
                 pl.BlockSpec((_BSN, 8), lambda s, i, _nb=nbn: (i + _nb, 0)),
                 pl.BlockSpec((_BSN, 7), lambda s, i: (i, 0)),
                 pl.BlockSpec((_BSN, 3), lambda s, i: (i, 0))]
                + _full_specs(arrs[4:]))
    return pl.pallas_call(
        functools.partial(_node2_body, float(n)),
        grid=(5, nbn),
        in_specs=in_specs,
        out_specs=[pl.BlockSpec((_BSN, 16), lambda s, i: (i, 0)),
                   pl.BlockSpec((_BSN, 16), lambda s, i: (i, 0))],
        out_shape=[jax.ShapeDtypeStruct((n, 16), F32),
                   jax.ShapeDtypeStruct((n, 16), F32)],
        scratch_shapes=[pltpu.VMEM((8, 8), F32), pltpu.VMEM((8, 64), F32)],
        compiler_params=pltpu.CompilerParams(
            dimension_semantics=("arbitrary", "arbitrary")),
    )(*arrs)


def _node3_body(ntot, *refs):
    at_ref, ab_ref = refs[0], refs[1]
    w = [refs[2 + j][...] for j in range(10)]
    g13, b13 = refs[12][...], refs[13][...]
    g14, b14 = refs[14][...], refs[15][...]
    wc2, bc2 = refs[16][...], refs[17][...]
    wl, bl = refs[18][...], refs[19][...]
    out = refs[20]
    st32, st64 = refs[21], refs[22]
    s, i = pl.program_id(0), pl.program_id(1)

    @pl.when(jnp.logical_and(s == 0, i == 0))
    def _():
        st32[...] = jnp.zeros_like(st32)
        st64[...] = jnp.zeros_like(st64)

    acc = at_ref[...] + ab_ref[...]
    out2 = _dot(acc[:, 1:8] / (acc[:, 0:1] + 1e-16), wc2) + bc2

    @pl.when(s == 0)
    def _():
        _acc_stats(st32, 0, out2)
        out[...] = jnp.zeros_like(out)

    @pl.when(jnp.logical_and(s == 1, i == 0))
    def _():
        _fin_stats(st32, 0, 4, ntot, 16)

    @pl.when(s >= 1)
    def _():
        z = _apply_bn(st32, 4, out2, g13, b13)
        t1 = _dot(z, w[0]) + w[1]

        @pl.when(s == 1)
        def _():
            _acc_stats(st64, 0, t1)
            out[...] = jnp.zeros_like(out)

        @pl.when(jnp.logical_and(s == 2, i == 0))
        def _():
            _fin_stats(st64, 0, 4, ntot, 64)

        @pl.when(s >= 2)
        def _():
            a1 = jnp.maximum(_apply_bn(st64, 4, t1, w[2], w[3]), 0.0)
            t2 = _dot(a1, w[4]) + w[5]

            @pl.when(s == 2)
            def _():
                _acc_stats(st64, 2, t2)
                out[...] = jnp.zeros_like(out)

            @pl.when(jnp.logical_and(s == 3, i == 0))
            def _():
                _fin_stats(st64, 2, 6, ntot, 64)

            @pl.when(s >= 3)
            def _():
                a2 = jnp.maximum(_apply_bn(st64, 6, t2, w[6], w[7]), 0.0)
                rh = jnp.maximum(_dot(a2, w[8]) + w[9], 0.0)

                @pl.when(s == 3)
                def _():
                    _acc_stats(st32, 8, rh)
                    out[...] = jnp.zeros_like(out)

                @pl.when(jnp.logical_and(s == 4, i == 0))
                def _():
                    _fin_stats(st32, 8, 10, ntot, 32)

                @pl.when(s == 4)
                def _():
                    hh = _apply_bn(st32, 10, rh, g14, b14)
                    zz = _dot(hh, wl) + bl
                    out[...] = 1.0 / (1.0 + jnp.exp(-zz))


def _node3(accp, p, n):
    g2 = p["conv1_2"]
    arrs = ([accp, accp] + _mlp_arrs(p["mlp1_3"])
            + [p["bn1_3"]["g"][None, :], p["bn1_3"]["b"][None, :],
               p["bn1_4"]["g"][None, :], p["bn1_4"]["b"][None, :],
               g2["W"], g2["b"][None, :],
               p["lin1_1"]["W"], p["lin1_1"]["b"][None, :]])
    nbn = n // _BSN
    in_specs = ([pl.BlockSpec((_BSN, 8), lambda s, i: (i, 0)),
                 pl.BlockSpec((_BSN, 8), lambda s, i, _nb=nbn: (i + _nb, 0))]
                + _full_specs(arrs[2:]))
    return pl.pallas_call(
        functools.partial(_node3_body, float(n)),
        grid=(5, nbn),
        in_specs=in_specs,
        out_specs=pl.BlockSpec((_BSN, 1), lambda s, i: (i, 0)),
        out_shape=jax.ShapeDtypeStruct((n, 1), F32),
        scratch_shapes=[pltpu.VMEM((12, 32), F32), pltpu.VMEM((8, 64), F32)],
        compiler_params=pltpu.CompilerParams(
            dimension_semantics=("arbitrary", "arbitrary")),
    )(*arrs)


def _edge_body(e_total, k, dh, *refs):
    gs_ref, gd_ref = refs[0], refs[1]
    w = [refs[2 + j][...] for j in range(10)]
    we, ae = refs[12][...], refs[13][...]
    msg_ref, st = refs[14], refs[15]
    s, i = pl.program_id(0), pl.program_id(1)

    @pl.when(jnp.logical_and(s == 0, i == 0))
    def _():
        st[...] = jnp.zeros_like(st)

    gs = gs_ref[...]
    gd = gd_ref[...]
    r = jnp.concatenate([gs[:, 0:3] - gd[:, 0:3], gd[:, 4:4 + k]], axis=1)
    t1 = _dot(r, w[0]) + w[1]

    @pl.when(s == 0)
    def _():
        _acc_stats(st, 0, t1)
        msg_ref[...] = jnp.zeros_like(msg_ref)

    @pl.when(jnp.logical_and(s == 1, i == 0))
    def _():
        _fin_stats(st, 0, 4, e_total, dh)

    @pl.when(s >= 1)
    def _():
        a1 = jnp.maximum(_apply_bn(st, 4, t1, w[2], w[3]), 0.0)
        t2 = _dot(a1, w[4]) + w[5]

        @pl.when(s == 1)
        def _():
            _acc_stats(st, 2, t2)
            msg_ref[...] = jnp.zeros_like(msg_ref)

        @pl.when(jnp.logical_and(s == 2, i == 0))
        def _():
            _fin_stats(st, 2, 6, e_total, dh)

        @pl.when(s == 2)
        def _():
            a2 = jnp.maximum(_apply_bn(st, 6, t2, w[6], w[7]), 0.0)
            eattr = jnp.maximum(_dot(a2, w[8]) + w[9], 0.0)
            ev = _dot(eattr, we)
            lo = gs[:, 3] + gd[:, 3] + jnp.sum(ev * ae, axis=1)
            lo = jnp.where(lo >= 0, lo, 0.2 * lo)
            wt = jnp.exp(lo)[:, None]
            msg_ref[...] = jnp.concatenate([wt, wt * gs[:, 4:11]], axis=1)


def _edge_stage(gs, gd, mlp_p, we, ae, k, dh):
    e = gs.shape[0]
    nb = e // _BSE
    arrs = [gs, gd] + _mlp_arrs(mlp_p) + [we, ae[None, :]]
    in_specs = ([pl.BlockSpec((_BSE, gs.shape[1]), lambda s, i: (i, 0)),
                 pl.BlockSpec((_BSE, gd.shape[1]), lambda s, i: (i, 0))]
                + _full_specs(arrs[2:]))
    return pl.pallas_call(
        functools.partial(_edge_body, float(e), k, dh),
        grid=(3, nb),
        in_specs=in_specs,
        out_specs=pl.BlockSpec((_BSE, 8), lambda s, i: (i, 0)),
        out_shape=jax.ShapeDtypeStruct((e, 8), F32),
        scratch_shapes=[pltpu.VMEM((8, dh), F32)],
        compiler_params=pltpu.CompilerParams(
            dimension_semantics=("arbitrary", "arbitrary")),
    )(*arrs)


_NW = 32


def _sc_mesh():
    return plsc.VectorSubcoreMesh(core_axis_name="c", subcore_axis_name="s")


def _sc_gather(tsrc, tdst, src, dst):
    n, d = tsrc.shape
    e = src.shape[0]
    ew = e // _NW
    cg = 5000
    nchunk = ew // cg

    @functools.partial(
        pl.kernel,
        mesh=_sc_mesh(),
        out_type=[jax.ShapeDtypeStruct((e, d), F32),
                  jax.ShapeDtypeStruct((e, d), F32)],
        scratch_types=[pltpu.VMEM((cg,), jnp.int32),
                       pltpu.VMEM((cg, d), F32),
                       pltpu.SemaphoreType.DMA],
        compiler_params=pltpu.CompilerParams(use_tc_tiling_on_sc=False),
    )
    def gk(ts_h, td_h, src_h, dst_h, gs_h, gd_h, idx_v, rows_v, sem):
        wid = lax.axis_index("s") * 2 + lax.axis_index("c")
        base_w = wid * ew
        for c in range(nchunk):
            base = base_w + c * cg
            pltpu.sync_copy(src_h.at[pl.ds(base, cg)], idx_v)
            pltpu.async_copy(ts_h.at[idx_v], rows_v, sem).wait()
            pltpu.sync_copy(rows_v, gs_h.at[pl.ds(base, cg)])
            pltpu.sync_copy(dst_h.at[pl.ds(base, cg)], idx_v)
            pltpu.async_copy(td_h.at[idx_v], rows_v, sem).wait()
            pltpu.sync_copy(rows_v, gd_h.at[pl.ds(base, cg)])

    return gk(tsrc, tdst, src, dst)


def _sc_scatter(msg, dst2d, zrs):
    e = msg.shape[0]
    n = zrs.shape[0]
    ew = e // _NW
    cg = 5000
    nchunk = ew // cg
    jrows = cg // 125
    npt = n // 16

    @functools.partial(
        pl.kernel,
        mesh=_sc_mesh(),
        out_type=jax.ShapeDtypeStruct((2 * n, 8), F32),
        scratch_types=[pltpu.VMEM_SHARED((n, 8), F32),
                       pltpu.VMEM((npt, 8), F32),
                       pltpu.VMEM((cg, 8), F32),
                       pltpu.VMEM((jrows, 125), jnp.int32)],
        compiler_params=pltpu.CompilerParams(use_tc_tiling_on_sc=False),
    )
    def sk(msg_h, dst_h, zrs_h, out_h, acc, zb, mb, ib):
        cid = lax.axis_index("c")
        sid = lax.axis_index("s")
        wid = sid * 2 + cid
        pltpu.sync_copy(zrs_h.at[pl.ds(sid * npt, npt)], zb)
        pltpu.sync_copy(zb, acc.at[pl.ds(sid * npt, npt)])
        plsc.subcore_barrier()
        for c in range(nchunk):
            base = wid * ew + c * cg
            pltpu.sync_copy(msg_h.at[pl.ds(base, cg)], mb)
            pltpu.sync_copy(dst_h.at[pl.ds(base // 125, jrows)], ib)

            def scat(j, carry):
                pltpu.sync_copy(mb.at[pl.ds(j * 125, 125)],
                                acc.at[ib.at[j]], add=True)
                return carry

            lax.fori_loop(0, jrows, scat, 0)
        plsc.subcore_barrier()
        pltpu.sync_copy(acc.at[pl.ds(sid * npt, npt)], zb)
        pltpu.sync_copy(zb, out_h.at[pl.ds(cid * n + sid * npt, npt)])

    return sk(msg, dst2d, zrs)


def kernel(x, edge_index, params):
    n = x.shape[0]
    e = edge_index.shape[1]
    p = params
    src = edge_index[0]
    dst = edge_index[1]
    dst2d = dst.reshape(e // 125, 125)
    zrs = jnp.zeros((n, 8), F32)
    g1 = p["conv1_1"]
    g2 = p["conv1_2"]

    src1, dst1, d2 = _node1(x, p, n)

    gs1, gd1 = _sc_gather(src1, dst1, src, dst)
    msg1 = _edge_stage(gs1, gd1, p["mlp1_4"], g1["We"], g1["ae"], 1, 64)
    accp1 = _sc_scatter(msg1, dst2d, zrs)

    src2, dst2 = _node2(accp1, x, d2, p, n)

    gs2, gd2 = _sc_gather(src2, dst2, src, dst)
    msg2 = _edge_stage(gs2, gd2, p["mlp1_8"], g2["We"], g2["ae"], 7, 32)
    accp2 = _sc_scatter(msg2, dst2d, zrs)

    return _node3(accp2, p, n)


# speedup vs baseline: 13.3409x; 1.1517x over previous
import functools

import jax
import jax.numpy as jnp
from jax import lax
from jax.experimental import pallas as pl
from jax.experimental.pallas import tpu as pltpu
from jax.experimental.pallas import tpu_sc as plsc

F32 = jnp.float32
_BSN = 5000
_BSE = 8000


def _mlp_arrs(p):
    ls, bs = p["lins"], p["bns"]
    return [ls[0]["W"], ls[0]["b"][None, :], bs[0]["g"][None, :], bs[0]["b"][None, :],
            ls[1]["W"], ls[1]["b"][None, :], bs[1]["g"][None, :], bs[1]["b"][None, :],
            ls[2]["W"], ls[2]["b"][None, :]]


def _acc_stats(st, r0, z):
    st[r0:r0 + 1, :z.shape[1]] += jnp.sum(z, axis=0, keepdims=True)
    st[r0 + 1:r0 + 2, :z.shape[1]] += jnp.sum(z * z, axis=0, keepdims=True)


def _fin_stats(st, r0, rdst, ntot, width):
    mu = st[r0:r0 + 1, :width] * (1.0 / ntot)
    var = st[r0 + 1:r0 + 2, :width] * (1.0 / ntot) - mu * mu
    st[rdst:rdst + 1, :width] = mu
    st[rdst + 1:rdst + 2, :width] = lax.rsqrt(var + 1e-5)


def _apply_bn(st, rdst, z, g, b):
    w = z.shape[1]
    return (z - st[rdst:rdst + 1, :w]) * st[rdst + 1:rdst + 2, :w] * g + b


def _dot(a, b):
    return jnp.dot(a, b, preferred_element_type=F32)


def _full_specs(arrs):
    return [pl.BlockSpec(a.shape, lambda s, i: (0, 0)) for a in arrs]


def _node1_body(ntot, *refs):
    x_ref = refs[0]
    wa = [refs[1 + j][...] for j in range(10)]
    wb = [refs[11 + j][...] for j in range(10)]
    wc1, asv, adv = refs[21][...], refs[22][...], refs[23][...]
    src1, dst1, d2o = refs[24], refs[25], refs[26]
    sta, stb = refs[27], refs[28]
    s, i = pl.program_id(0), pl.program_id(1)

    @pl.when(jnp.logical_and(s == 0, i == 0))
    def _():
        sta[...] = jnp.zeros_like(sta)
        stb[...] = jnp.zeros_like(stb)

    x = x_ref[...]
    x3 = x[:, 0:3]
    t1a = _dot(x3, wa[0]) + wa[1]
    t1b = _dot(x3, wb[0]) + wb[1]

    @pl.when(s == 0)
    def _():
        _acc_stats(sta, 0, t1a)
        _acc_stats(stb, 0, t1b)
        src1[...] = jnp.zeros_like(src1)
        dst1[...] = jnp.zeros_like(dst1)
        d2o[...] = jnp.zeros_like(d2o)

    @pl.when(jnp.logical_and(s == 1, i == 0))
    def _():
        _fin_stats(sta, 0, 4, ntot, 64)
        _fin_stats(stb, 0, 4, ntot, 32)

    @pl.when(s >= 1)
    def _():
        a1a = jnp.maximum(_apply_bn(sta, 4, t1a, wa[2], wa[3]), 0.0)
        a1b = jnp.maximum(_apply_bn(stb, 4, t1b, wb[2], wb[3]), 0.0)
        t2a = _dot(a1a, wa[4]) + wa[5]
        t2b = _dot(a1b, wb[4]) + wb[5]

        @pl.when(s == 1)
        def _():
            _acc_stats(sta, 2, t2a)
            _acc_stats(stb, 2, t2b)
            src1[...] = jnp.zeros_like(src1)
            dst1[...] = jnp.zeros_like(dst1)
            d2o[...] = jnp.zeros_like(d2o)

        @pl.when(jnp.logical_and(s == 2, i == 0))
        def _():
            _fin_stats(sta, 2, 6, ntot, 64)
            _fin_stats(stb, 2, 6, ntot, 32)

        @pl.when(s == 2)
        def _():
            a2a = jnp.maximum(_apply_bn(sta, 6, t2a, wa[6], wa[7]), 0.0)
            a2b = jnp.maximum(_apply_bn(stb, 6, t2b, wb[6], wb[7]), 0.0)
            d1 = _dot(a2a, wa[8]) + wa[9]
            d2 = _dot(a2b, wb[8]) + wb[9]
            h0 = _dot(x, wc1)
            sa = jnp.sum(h0 * asv, axis=1, keepdims=True)
            sd = jnp.sum(h0 * adv, axis=1, keepdims=True)
            bm = x.shape[0]
            src1[...] = jnp.concatenate(
                [x3 + d1, sa, x, jnp.zeros((bm, 5), F32)], axis=1)
            dst1[...] = jnp.concatenate(
                [x3, sd, x[:, 3:4], jnp.zeros((bm, 11), F32)], axis=1)
            d2o[...] = d2


def _node1(x, p, n):
    g1 = p["conv1_1"]
    arrs = ([x] + _mlp_arrs(p["mlp1_1"]) + _mlp_arrs(p["mlp1_7"])
            + [g1["W"], g1["as"][None, :], g1["ad"][None, :]])
    nbn = n // _BSN
    in_specs = ([pl.BlockSpec((_BSN, 7), lambda s, i: (i, 0))]
                + _full_specs(arrs[1:]))
    return pl.pallas_call(
        functools.partial(_node1_body, float(n)),
        grid=(3, nbn),
        in_specs=in_specs,
        out_specs=[pl.BlockSpec((_BSN, 16), lambda s, i: (i, 0)),
                   pl.BlockSpec((_BSN, 16), lambda s, i: (i, 0)),
                   pl.BlockSpec((_BSN, 3), lambda s, i: (i, 0))],
        out_shape=[jax.ShapeDtypeStruct((n, 16), F32),
                   jax.ShapeDtypeStruct((n, 16), F32),
                   jax.ShapeDtypeStruct((n, 3), F32)],
        scratch_shapes=[pltpu.VMEM((8, 64), F32), pltpu.VMEM((8, 32), F32)],
        compiler_params=pltpu.CompilerParams(
            dimension_semantics=("arbitrary", "arbitrary")),
    )(*arrs)


def _node2_body(ntot, *refs):
    at_ref, ab_ref, x_ref, d2_ref = refs[0], refs[1], refs[2], refs[3]
    w = [refs[4 + j][...] for j in range(10)]
    g11, b11 = refs[14][...], refs[15][...]
    g12, b12 = refs[16][...], refs[17][...]
    wc1, bc1 = refs[18][...], refs[19][...]
    wc2, as2, ad2 = refs[20][...], refs[21][...], refs[22][...]
    src2, dst2 = refs[23], refs[24]
    st7, st64 = refs[25], refs[26]
    s, i = pl.program_id(0), pl.program_id(1)

    @pl.when(jnp.logical_and(s == 0, i == 0))
    def _():
        st7[...] = jnp.zeros_like(st7)
        st64[...] = jnp.zeros_like(st64)

    acc = at_ref[...] + ab_ref[...]
    out1 = _dot(acc[:, 1:8] / (acc[:, 0:1] + 1e-16), wc1) + bc1

    def wzero():
        src2[...] = jnp.zeros_like(src2)
        dst2[...] = jnp.zeros_like(dst2)

    @pl.when(s == 0)
    def _():
        _acc_stats(st7, 0, out1)
        wzero()

    @pl.when(jnp.logical_and(s == 1, i == 0))
    def _():
        _fin_stats(st7, 0, 4, ntot, 7)

    @pl.when(s >= 1)
    def _():
        z = _apply_bn(st7, 4, out1, g11, b11)
        t1 = _dot(z, w[0]) + w[1]

        @pl.when(s == 1)
        def _():
            _acc_stats(st64, 0, t1)
            wzero()

        @pl.when(jnp.logical_and(s == 2, i == 0))
        def _():
            _fin_stats(st64, 0, 4, ntot, 64)

        @pl.when(s >= 2)
        def _():
            a1 = jnp.maximum(_apply_bn(st64, 4, t1, w[2], w[3]), 0.0)
            t2 = _dot(a1, w[4]) + w[5]

            @pl.when(s == 2)
            def _():
                _acc_stats(st64, 2, t2)
                wzero()

            @pl.when(jnp.logical_and(s == 3, i == 0))
            def _():
                _fin_stats(st64, 2, 6, ntot, 64)

            @pl.when(s >= 3)
            def _():
                a2 = jnp.maximum(_apply_bn(st64, 6, t2, w[6], w[7]), 0.0)
                rh = jnp.maximum(_dot(a2, w[8]) + w[9], 0.0)

                @pl.when(s == 3)
                def _():
                    _acc_stats(st7, 2, rh)
                    wzero()

                @pl.when(jnp.logical_and(s == 4, i == 0))
                def _():
                    _fin_stats(st7, 2, 6, ntot, 7)

                @pl.when(s == 4)
                def _():
                    h = _apply_bn(st7, 6, rh, g12, b12)
                    h2 = _dot(h, wc2)
                    sa = jnp.sum(h2 * as2, axis=1, keepdims=True)
                    sd = jnp.sum(h2 * ad2, axis=1, keepdims=True)
                    x = x_ref[...]
                    bm = x.shape[0]
                    pad = jnp.zeros((bm, 5), F32)
                    src2[...] = jnp.concatenate(
                        [x[:, 0:3] + d2_ref[...], sa, h, pad], axis=1)
                    dst2[...] = jnp.concatenate(
                        [x[:, 0:3], sd, h, pad], axis=1)


def _node2(accp, x, d2, p, n):
    g1, g2 = p["conv1_1"], p["conv1_2"]
    arrs = ([accp, accp, x, d2] + _mlp_arrs(p["mlp1_2"])
            + [p["bn1_1"]["g"][None, :], p["bn1_1"]["b"][None, :],
               p["bn1_2"]["g"][None, :], p["bn1_2"]["b"][None, :],
               g1["W"], g1["b"][None, :],
               g2["W"], g2["as"][None, :], g2["ad"][None, :]])
    nbn = n // _BSN
    in_specs = ([pl.BlockSpec((_BSN, 8), lambda s, i: (i, 0)),
                 pl.BlockSpec((_BSN, 8), lambda s, i, _nb=nbn: (i + _nb, 0)),
                 pl.BlockSpec((_BSN, 7), lambda s, i: (i, 0)),
                 pl.BlockSpec((_BSN, 3), lambda s, i: (i, 0))]
                + _full_specs(arrs[4:]))
    return pl.pallas_call(
        functools.partial(_node2_body, float(n)),
        grid=(5, nbn),
        in_specs=in_specs,
        out_specs=[pl.BlockSpec((_BSN, 16), lambda s, i: (i, 0)),
                   pl.BlockSpec((_BSN, 16), lambda s, i: (i, 0))],
        out_shape=[jax.ShapeDtypeStruct((n, 16), F32),
                   jax.ShapeDtypeStruct((n, 16), F32)],
        scratch_shapes=[pltpu.VMEM((8, 8), F32), pltpu.VMEM((8, 64), F32)],
        compiler_params=pltpu.CompilerParams(
            dimension_semantics=("arbitrary", "arbitrary")),
    )(*arrs)


def _node3_body(ntot, *refs):
    at_ref, ab_ref = refs[0], refs[1]
    w = [refs[2 + j][...] for j in range(10)]
    g13, b13 = refs[12][...], refs[13][...]
    g14, b14 = refs[14][...], refs[15][...]
    wc2, bc2 = refs[16][...], refs[17][...]
    wl, bl = refs[18][...], refs[19][...]
    out = refs[20]
    st32, st64 = refs[21], refs[22]
    s, i = pl.program_id(0), pl.program_id(1)

    @pl.when(jnp.logical_and(s == 0, i == 0))
    def _():
        st32[...] = jnp.zeros_like(st32)
        st64[...] = jnp.zeros_like(st64)

    acc = at_ref[...] + ab_ref[...]
    out2 = _dot(acc[:, 1:8] / (acc[:, 0:1] + 1e-16), wc2) + bc2

    @pl.when(s == 0)
    def _():
        _acc_stats(st32, 0, out2)
        out[...] = jnp.zeros_like(out)

    @pl.when(jnp.logical_and(s == 1, i == 0))
    def _():
        _fin_stats(st32, 0, 4, ntot, 16)

    @pl.when(s >= 1)
    def _():
        z = _apply_bn(st32, 4, out2, g13, b13)
        t1 = _dot(z, w[0]) + w[1]

        @pl.when(s == 1)
        def _():
            _acc_stats(st64, 0, t1)
            out[...] = jnp.zeros_like(out)

        @pl.when(jnp.logical_and(s == 2, i == 0))
        def _():
            _fin_stats(st64, 0, 4, ntot, 64)

        @pl.when(s >= 2)
        def _():
            a1 = jnp.maximum(_apply_bn(st64, 4, t1, w[2], w[3]), 0.0)
            t2 = _dot(a1, w[4]) + w[5]

            @pl.when(s == 2)
            def _():
                _acc_stats(st64, 2, t2)
                out[...] = jnp.zeros_like(out)

            @pl.when(jnp.logical_and(s == 3, i == 0))
            def _():
                _fin_stats(st64, 2, 6, ntot, 64)

            @pl.when(s >= 3)
            def _():
                a2 = jnp.maximum(_apply_bn(st64, 6, t2, w[6], w[7]), 0.0)
                rh = jnp.maximum(_dot(a2, w[8]) + w[9], 0.0)

                @pl.when(s == 3)
                def _():
                    _acc_stats(st32, 8, rh)
                    out[...] = jnp.zeros_like(out)

                @pl.when(jnp.logical_and(s == 4, i == 0))
                def _():
                    _fin_stats(st32, 8, 10, ntot, 32)

                @pl.when(s == 4)
                def _():
                    hh = _apply_bn(st32, 10, rh, g14, b14)
                    zz = _dot(hh, wl) + bl
                    out[...] = 1.0 / (1.0 + jnp.exp(-zz))


def _node3(accp, p, n):
    g2 = p["conv1_2"]
    arrs = ([accp, accp] + _mlp_arrs(p["mlp1_3"])
            + [p["bn1_3"]["g"][None, :], p["bn1_3"]["b"][None, :],
               p["bn1_4"]["g"][None, :], p["bn1_4"]["b"][None, :],
               g2["W"], g2["b"][None, :],
               p["lin1_1"]["W"], p["lin1_1"]["b"][None, :]])
    nbn = n // _BSN
    in_specs = ([pl.BlockSpec((_BSN, 8), lambda s, i: (i, 0)),
                 pl.BlockSpec((_BSN, 8), lambda s, i, _nb=nbn: (i + _nb, 0))]
                + _full_specs(arrs[2:]))
    return pl.pallas_call(
        functools.partial(_node3_body, float(n)),
        grid=(5, nbn),
        in_specs=in_specs,
        out_specs=pl.BlockSpec((_BSN, 1), lambda s, i: (i, 0)),
        out_shape=jax.ShapeDtypeStruct((n, 1), F32),
        scratch_shapes=[pltpu.VMEM((12, 32), F32), pltpu.VMEM((8, 64), F32)],
        compiler_params=pltpu.CompilerParams(
            dimension_semantics=("arbitrary", "arbitrary")),
    )(*arrs)


def _edge_body(e_total, k, dh, *refs):
    gs_ref, gd_ref = refs[0], refs[1]
    w = [refs[2 + j][...] for j in range(10)]
    we, ae = refs[12][...], refs[13][...]
    msg_ref, st = refs[14], refs[15]
    s, i = pl.program_id(0), pl.program_id(1)

    @pl.when(jnp.logical_and(s == 0, i == 0))
    def _():
        st[...] = jnp.zeros_like(st)

    gs = gs_ref[...]
    gd = gd_ref[...]
    r = jnp.concatenate([gs[:, 0:3] - gd[:, 0:3], gd[:, 4:4 + k]], axis=1)
    t1 = _dot(r, w[0]) + w[1]

    @pl.when(s == 0)
    def _():
        _acc_stats(st, 0, t1)
        msg_ref[...] = jnp.zeros_like(msg_ref)

    @pl.when(jnp.logical_and(s == 1, i == 0))
    def _():
        _fin_stats(st, 0, 4, e_total, dh)

    @pl.when(s >= 1)
    def _():
        a1 = jnp.maximum(_apply_bn(st, 4, t1, w[2], w[3]), 0.0)
        t2 = _dot(a1, w[4]) + w[5]

        @pl.when(s == 1)
        def _():
            _acc_stats(st, 2, t2)
            msg_ref[...] = jnp.zeros_like(msg_ref)

        @pl.when(jnp.logical_and(s == 2, i == 0))
        def _():
            _fin_stats(st, 2, 6, e_total, dh)

        @pl.when(s == 2)
        def _():
            a2 = jnp.maximum(_apply_bn(st, 6, t2, w[6], w[7]), 0.0)
            eattr = jnp.maximum(_dot(a2, w[8]) + w[9], 0.0)
            ev = _dot(eattr, we)
            lo = gs[:, 3] + gd[:, 3] + jnp.sum(ev * ae, axis=1)
            lo = jnp.where(lo >= 0, lo, 0.2 * lo)
            wt = jnp.exp(lo)[:, None]
            msg_ref[...] = jnp.concatenate([wt, wt * gs[:, 4:11]], axis=1)


def _edge_stage(gs, gd, mlp_p, we, ae, k, dh):
    e = gs.shape[0]
    nb = e // _BSE
    arrs = [gs, gd] + _mlp_arrs(mlp_p) + [we, ae[None, :]]
    in_specs = ([pl.BlockSpec((_BSE, gs.shape[1]), lambda s, i: (i, 0)),
                 pl.BlockSpec((_BSE, gd.shape[1]), lambda s, i: (i, 0))]
                + _full_specs(arrs[2:]))
    return pl.pallas_call(
        functools.partial(_edge_body, float(e), k, dh),
        grid=(3, nb),
        in_specs=in_specs,
        out_specs=pl.BlockSpec((_BSE, 8), lambda s, i: (i, 0)),
        out_shape=jax.ShapeDtypeStruct((e, 8), F32),
        scratch_shapes=[pltpu.VMEM((8, dh), F32)],
        compiler_params=pltpu.CompilerParams(
            dimension_semantics=("arbitrary", "arbitrary")),
    )(*arrs)


_NW = 32


def _sc_mesh():
    return plsc.VectorSubcoreMesh(core_axis_name="c", subcore_axis_name="s")


def _sc_gather(tsrc, tdst, src, dst):
    n, d = tsrc.shape
    e = src.shape[0]
    ew = e // _NW
    cg = 5000
    nchunk = ew // cg

    @functools.partial(
        pl.kernel,
        mesh=_sc_mesh(),
        out_type=[jax.ShapeDtypeStruct((e, d), F32),
                  jax.ShapeDtypeStruct((e, d), F32)],
        scratch_types=[pltpu.VMEM((cg,), jnp.int32),
                       pltpu.VMEM((cg, d), F32),
                       pltpu.SemaphoreType.DMA],
        compiler_params=pltpu.CompilerParams(use_tc_tiling_on_sc=False),
    )
    def gk(ts_h, td_h, src_h, dst_h, gs_h, gd_h, idx_v, rows_v, sem):
        wid = lax.axis_index("s") * 2 + lax.axis_index("c")
        base_w = wid * ew
        for c in range(nchunk):
            base = base_w + c * cg
            pltpu.sync_copy(src_h.at[pl.ds(base, cg)], idx_v)
            pltpu.async_copy(ts_h.at[idx_v], rows_v, sem).wait()
            pltpu.sync_copy(rows_v, gs_h.at[pl.ds(base, cg)])
            pltpu.sync_copy(dst_h.at[pl.ds(base, cg)], idx_v)
            pltpu.async_copy(td_h.at[idx_v], rows_v, sem).wait()
            pltpu.sync_copy(rows_v, gd_h.at[pl.ds(base, cg)])

    return gk(tsrc, tdst, src, dst)


def _sc_scatter(msg, dst2d, zrs):
    e = msg.shape[0]
    n = zrs.shape[0]
    ew = e // _NW
    cg = 5000
    nchunk = ew // cg
    jrows = cg // 125
    npt = n // 16

    @functools.partial(
        pl.kernel,
        mesh=_sc_mesh(),
        out_type=jax.ShapeDtypeStruct((2 * n, 8), F32),
        scratch_types=[pltpu.VMEM_SHARED((n, 8), F32),
                       pltpu.VMEM((npt, 8), F32),
                       pltpu.VMEM((cg, 8), F32),
                       pltpu.VMEM((jrows, 125), jnp.int32)],
        compiler_params=pltpu.CompilerParams(use_tc_tiling_on_sc=False),
    )
    def sk(msg_h, dst_h, zrs_h, out_h, acc, zb, mb, ib):
        cid = lax.axis_index("c")
        sid = lax.axis_index("s")
        wid = sid * 2 + cid
        pltpu.sync_copy(zrs_h.at[pl.ds(sid * npt, npt)], zb)
        pltpu.sync_copy(zb, acc.at[pl.ds(sid * npt, npt)])
        plsc.subcore_barrier()
        for c in range(nchunk):
            base = wid * ew + c * cg
            pltpu.sync_copy(msg_h.at[pl.ds(base, cg)], mb)
            pltpu.sync_copy(dst_h.at[pl.ds(base // 125, jrows)], ib)

            def scat(j, carry):
                pltpu.sync_copy(mb.at[pl.ds(j * 125, 125)],
                                acc.at[ib.at[j]], add=True)
                return carry

            lax.fori_loop(0, jrows, scat, 0)
        plsc.subcore_barrier()
        pltpu.sync_copy(acc.at[pl.ds(sid * npt, npt)], zb)
        pltpu.sync_copy(zb, out_h.at[pl.ds(cid * n + sid * npt, npt)])

    return sk(msg, dst2d, zrs)


def kernel(x, edge_index, params):
    n = x.shape[0]
    e = edge_index.shape[1]
    p = params
    src = edge_index[0]
    dst = edge_index[1]
    dst2d = dst.reshape(e // 125, 125)
    zrs = jnp.zeros((n, 8), F32)
    g1 = p["conv1_1"]
    g2 = p["conv1_2"]

    src1, dst1, d2 = _node1(x, p, n)

    gs1, gd1 = _sc_gather(src1, dst1, src, dst)
    msg1 = _edge_stage(gs1, gd1, p["mlp1_4"], g1["We"], g1["ae"], 1, 64)
    accp1 = _sc_scatter(msg1, dst2d, zrs)

    src2, dst2 = _node2(accp1, x, d2, p, n)

    gs2, gd2 = _sc_gather(src2, dst2, src, dst)
    msg2 = _edge_stage(gs2, gd2, p["mlp1_8"], g2["We"], g2["ae"], 7, 32)
    accp2 = _sc_scatter(msg2, dst2d, zrs)

    return _node3(accp2, p, n)


# speedup vs baseline: 13.4664x; 1.0094x over previous
import functools

import jax
import jax.numpy as jnp
from jax import lax
from jax.experimental import pallas as pl
from jax.experimental.pallas import tpu as pltpu
from jax.experimental.pallas import tpu_sc as plsc

F32 = jnp.float32
_BSN = 5000
_BSE = 8000


def _mlp_arrs(p):
    ls, bs = p["lins"], p["bns"]
    return [ls[0]["W"], ls[0]["b"][None, :], bs[0]["g"][None, :], bs[0]["b"][None, :],
            ls[1]["W"], ls[1]["b"][None, :], bs[1]["g"][None, :], bs[1]["b"][None, :],
            ls[2]["W"], ls[2]["b"][None, :]]


def _acc_stats(st, r0, z):
    ones = jnp.ones((1, z.shape[0]), F32)
    w = z.shape[1]
    st[r0:r0 + 1, :w] += jnp.dot(ones, z, preferred_element_type=F32)
    st[r0 + 1:r0 + 2, :w] += jnp.dot(ones, z * z, preferred_element_type=F32)


def _fin_stats(st, r0, rdst, ntot, width, g, b):
    mu = st[r0:r0 + 1, :width] * (1.0 / ntot)
    var = st[r0 + 1:r0 + 2, :width] * (1.0 / ntot) - mu * mu
    a = lax.rsqrt(var + 1e-5) * g[:, :width]
    st[rdst:rdst + 1, :width] = a
    st[rdst + 1:rdst + 2, :width] = b[:, :width] - mu * a


def _apply_bn(st, rdst, z):
    w = z.shape[1]
    return z * st[rdst:rdst + 1, :w] + st[rdst + 1:rdst + 2, :w]


def _dot(a, b):
    return jnp.dot(a, b, preferred_element_type=F32)


def _full_specs(arrs):
    return [pl.BlockSpec(a.shape, lambda s, i: (0, 0)) for a in arrs]


def _node1_body(ntot, *refs):
    x_ref = refs[0]
    wa = [refs[1 + j][...] for j in range(10)]
    wb = [refs[11 + j][...] for j in range(10)]
    wc1, asv, adv = refs[21][...], refs[22][...], refs[23][...]
    src1, dst1, d2o = refs[24], refs[25], refs[26]
    sta, stb = refs[27], refs[28]
    s, i = pl.program_id(0), pl.program_id(1)

    @pl.when(jnp.logical_and(s == 0, i == 0))
    def _():
        sta[...] = jnp.zeros_like(sta)
        stb[...] = jnp.zeros_like(stb)

    x = x_ref[...]
    x3 = x[:, 0:3]
    t1a = _dot(x3, wa[0]) + wa[1]
    t1b = _dot(x3, wb[0]) + wb[1]

    @pl.when(s == 0)
    def _():
        _acc_stats(sta, 0, t1a)
        _acc_stats(stb, 0, t1b)
        pass

    @pl.when(jnp.logical_and(s == 1, i == 0))
    def _():
        _fin_stats(sta, 0, 4, ntot, 64, wa[2], wa[3])
        _fin_stats(stb, 0, 4, ntot, 32, wb[2], wb[3])

    @pl.when(s >= 1)
    def _():
        a1a = jnp.maximum(_apply_bn(sta, 4, t1a), 0.0)
        a1b = jnp.maximum(_apply_bn(stb, 4, t1b), 0.0)
        t2a = _dot(a1a, wa[4]) + wa[5]
        t2b = _dot(a1b, wb[4]) + wb[5]

        @pl.when(s == 1)
        def _():
            _acc_stats(sta, 2, t2a)
            _acc_stats(stb, 2, t2b)
        pass

        @pl.when(jnp.logical_and(s == 2, i == 0))
        def _():
            _fin_stats(sta, 2, 6, ntot, 64, wa[6], wa[7])
            _fin_stats(stb, 2, 6, ntot, 32, wb[6], wb[7])

        @pl.when(s == 2)
        def _():
            a2a = jnp.maximum(_apply_bn(sta, 6, t2a), 0.0)
            a2b = jnp.maximum(_apply_bn(stb, 6, t2b), 0.0)
            d1 = _dot(a2a, wa[8]) + wa[9]
            d2 = _dot(a2b, wb[8]) + wb[9]
            h0 = _dot(x, wc1)
            sa = jnp.sum(h0 * asv, axis=1, keepdims=True)
            sd = jnp.sum(h0 * adv, axis=1, keepdims=True)
            bm = x.shape[0]
            src1[...] = jnp.concatenate(
                [x3 + d1, sa, x, jnp.zeros((bm, 5), F32)], axis=1)
            dst1[...] = jnp.concatenate(
                [x3, sd, x[:, 3:4], jnp.zeros((bm, 11), F32)], axis=1)
            d2o[...] = d2


def _node1(x, p, n):
    g1 = p["conv1_1"]
    arrs = ([x] + _mlp_arrs(p["mlp1_1"]) + _mlp_arrs(p["mlp1_7"])
            + [g1["W"], g1["as"][None, :], g1["ad"][None, :]])
    nbn = n // _BSN
    in_specs = ([pl.BlockSpec((_BSN, 7), lambda s, i: (i, 0))]
                + _full_specs(arrs[1:]))
    return pl.pallas_call(
        functools.partial(_node1_body, float(n)),
        grid=(3, nbn),
        in_specs=in_specs,
        out_specs=[pl.BlockSpec((_BSN, 16), lambda s, i: (i, 0)),
                   pl.BlockSpec((_BSN, 16), lambda s, i: (i, 0)),
                   pl.BlockSpec((_BSN, 3), lambda s, i: (i, 0))],
        out_shape=[jax.ShapeDtypeStruct((n, 16), F32),
                   jax.ShapeDtypeStruct((n, 16), F32),
                   jax.ShapeDtypeStruct((n, 3), F32)],
        scratch_shapes=[pltpu.VMEM((8, 64), F32), pltpu.VMEM((8, 32), F32)],
        compiler_params=pltpu.CompilerParams(
            dimension_semantics=("arbitrary", "arbitrary")),
    )(*arrs)


def _node2_body(ntot, *refs):
    at_ref, ab_ref, x_ref, d2_ref = refs[0], refs[1], refs[2], refs[3]
    w = [refs[4 + j][...] for j in range(10)]
    g11, b11 = refs[14][...], refs[15][...]
    g12, b12 = refs[16][...], refs[17][...]
    wc1, bc1 = refs[18][...], refs[19][...]
    wc2, as2, ad2 = refs[20][...], refs[21][...], refs[22][...]
    src2, dst2 = refs[23], refs[24]
    st7, st64 = refs[25], refs[26]
    s, i = pl.program_id(0), pl.program_id(1)

    @pl.when(jnp.logical_and(s == 0, i == 0))
    def _():
        st7[...] = jnp.zeros_like(st7)
        st64[...] = jnp.zeros_like(st64)

    acc = at_ref[...] + ab_ref[...]
    out1 = _dot(acc[:, 1:8] / (acc[:, 0:1] + 1e-16), wc1) + bc1

    def wzero():
        pass

    @pl.when(s == 0)
    def _():
        _acc_stats(st7, 0, out1)
        wzero()

    @pl.when(jnp.logical_and(s == 1, i == 0))
    def _():
        _fin_stats(st7, 0, 4, ntot, 7, g11, b11)

    @pl.when(s >= 1)
    def _():
        z = _apply_bn(st7, 4, out1)
        t1 = _dot(z, w[0]) + w[1]

        @pl.when(s == 1)
        def _():
            _acc_stats(st64, 0, t1)
            wzero()

        @pl.when(jnp.logical_and(s == 2, i == 0))
        def _():
            _fin_stats(st64, 0, 4, ntot, 64, w[2], w[3])

        @pl.when(s >= 2)
        def _():
            a1 = jnp.maximum(_apply_bn(st64, 4, t1), 0.0)
            t2 = _dot(a1, w[4]) + w[5]

            @pl.when(s == 2)
            def _():
                _acc_stats(st64, 2, t2)
                wzero()

            @pl.when(jnp.logical_and(s == 3, i == 0))
            def _():
                _fin_stats(st64, 2, 6, ntot, 64, w[6], w[7])

            @pl.when(s >= 3)
            def _():
                a2 = jnp.maximum(_apply_bn(st64, 6, t2), 0.0)
                rh = jnp.maximum(_dot(a2, w[8]) + w[9], 0.0)

                @pl.when(s == 3)
                def _():
                    _acc_stats(st7, 2, rh)
                    wzero()

                @pl.when(jnp.logical_and(s == 4, i == 0))
                def _():
                    _fin_stats(st7, 2, 6, ntot, 7, g12, b12)

                @pl.when(s == 4)
                def _():
                    h = _apply_bn(st7, 6, rh)
                    h2 = _dot(h, wc2)
                    sa = jnp.sum(h2 * as2, axis=1, keepdims=True)
                    sd = jnp.sum(h2 * ad2, axis=1, keepdims=True)
                    x = x_ref[...]
                    bm = x.shape[0]
                    pad = jnp.zeros((bm, 5), F32)
                    src2[...] = jnp.concatenate(
                        [x[:, 0:3] + d2_ref[...], sa, h, pad], axis=1)
                    dst2[...] = jnp.concatenate(
                        [x[:, 0:3], sd, h, pad], axis=1)


def _node2(accp, x, d2, p, n):
    g1, g2 = p["conv1_1"], p["conv1_2"]
    arrs = ([accp, accp, x, d2] + _mlp_arrs(p["mlp1_2"])
            + [p["bn1_1"]["g"][None, :], p["bn1_1"]["b"][None, :],
               p["bn1_2"]["g"][None, :], p["bn1_2"]["b"][None, :],
               g1["W"], g1["b"][None, :],
               g2["W"], g2["as"][None, :], g2["ad"][None, :]])
    nbn = n // _BSN
    in_specs = ([pl.BlockSpec((_BSN, 8), lambda s, i: (i, 0)),
                 pl.BlockSpec((_BSN, 8), lambda s, i, _nb=nbn: (i + _nb, 0)),
                 pl.BlockSpec((_BSN, 7), lambda s, i: (i, 0)),
                 pl.BlockSpec((_BSN, 3), lambda s, i: (i, 0))]
                + _full_specs(arrs[4:]))
    return pl.pallas_call(
        functools.partial(_node2_body, float(n)),
        grid=(5, nbn),
        in_specs=in_specs,
        out_specs=[pl.BlockSpec((_BSN, 16), lambda s, i: (i, 0)),
                   pl.BlockSpec((_BSN, 16), lambda s, i: (i, 0))],
        out_shape=[jax.ShapeDtypeStruct((n, 16), F32),
                   jax.ShapeDtypeStruct((n, 16), F32)],
        scratch_shapes=[pltpu.VMEM((8, 8), F32), pltpu.VMEM((8, 64), F32)],
        compiler_params=pltpu.CompilerParams(
            dimension_semantics=("arbitrary", "arbitrary")),
    )(*arrs)


def _node3_body(ntot, *refs):
    at_ref, ab_ref = refs[0], refs[1]
    w = [refs[2 + j][...] for j in range(10)]
    g13, b13 = refs[12][...], refs[13][...]
    g14, b14 = refs[14][...], refs[15][...]
    wc2, bc2 = refs[16][...], refs[17][...]
    wl, bl = refs[18][...], refs[19][...]
    out = refs[20]
    st32, st64 = refs[21], refs[22]
    s, i = pl.program_id(0), pl.program_id(1)

    @pl.when(jnp.logical_and(s == 0, i == 0))
    def _():
        st32[...] = jnp.zeros_like(st32)
        st64[...] = jnp.zeros_like(st64)

    acc = at_ref[...] + ab_ref[...]
    out2 = _dot(acc[:, 1:8] / (acc[:, 0:1] + 1e-16), wc2) + bc2

    @pl.when(s == 0)
    def _():
        _acc_stats(st32, 0, out2)

    @pl.when(jnp.logical_and(s == 1, i == 0))
    def _():
        _fin_stats(st32, 0, 4, ntot, 16, g13, b13)

    @pl.when(s >= 1)
    def _():
        z = _apply_bn(st32, 4, out2)
        t1 = _dot(z, w[0]) + w[1]

        @pl.when(s == 1)
        def _():
            _acc_stats(st64, 0, t1)

        @pl.when(jnp.logical_and(s == 2, i == 0))
        def _():
            _fin_stats(st64, 0, 4, ntot, 64, w[2], w[3])

        @pl.when(s >= 2)
        def _():
            a1 = jnp.maximum(_apply_bn(st64, 4, t1), 0.0)
            t2 = _dot(a1, w[4]) + w[5]

            @pl.when(s == 2)
            def _():
                _acc_stats(st64, 2, t2)

            @pl.when(jnp.logical_and(s == 3, i == 0))
            def _():
                _fin_stats(st64, 2, 6, ntot, 64, w[6], w[7])

            @pl.when(s >= 3)
            def _():
                a2 = jnp.maximum(_apply_bn(st64, 6, t2), 0.0)
                rh = jnp.maximum(_dot(a2, w[8]) + w[9], 0.0)

                @pl.when(s == 3)
                def _():
                    _acc_stats(st32, 8, rh)

                @pl.when(jnp.logical_and(s == 4, i == 0))
                def _():
                    _fin_stats(st32, 8, 10, ntot, 32, g14, b14)

                @pl.when(s == 4)
                def _():
                    hh = _apply_bn(st32, 10, rh)
                    zz = _dot(hh, wl) + bl
                    out[...] = 1.0 / (1.0 + jnp.exp(-zz))


def _node3(accp, p, n):
    g2 = p["conv1_2"]
    arrs = ([accp, accp] + _mlp_arrs(p["mlp1_3"])
            + [p["bn1_3"]["g"][None, :], p["bn1_3"]["b"][None, :],
               p["bn1_4"]["g"][None, :], p["bn1_4"]["b"][None, :],
               g2["W"], g2["b"][None, :],
               p["lin1_1"]["W"], p["lin1_1"]["b"][None, :]])
    nbn = n // _BSN
    in_specs = ([pl.BlockSpec((_BSN, 8), lambda s, i: (i, 0)),
                 pl.BlockSpec((_BSN, 8), lambda s, i, _nb=nbn: (i + _nb, 0))]
                + _full_specs(arrs[2:]))
    return pl.pallas_call(
        functools.partial(_node3_body, float(n)),
        grid=(5, nbn),
        in_specs=in_specs,
        out_specs=pl.BlockSpec((_BSN, 1), lambda s, i: (i, 0)),
        out_shape=jax.ShapeDtypeStruct((n, 1), F32),
        scratch_shapes=[pltpu.VMEM((12, 32), F32), pltpu.VMEM((8, 64), F32)],
        compiler_params=pltpu.CompilerParams(
            dimension_semantics=("arbitrary", "arbitrary")),
    )(*arrs)


def _edge_body(e_total, k, dh, *refs):
    gs_ref, gd_ref = refs[0], refs[1]
    w = [refs[2 + j][...] for j in range(10)]
    we, ae = refs[12][...], refs[13][...]
    msg_ref, st = refs[14], refs[15]
    s, i = pl.program_id(0), pl.program_id(1)
    rk = 3 + k

    @pl.when(jnp.logical_and(s == 0, i == 0))
    def _():
        st[...] = jnp.zeros_like(st)

    gs = gs_ref[...]
    gd = gd_ref[...]
    r = jnp.concatenate([gs[:, 0:3] - gd[:, 0:3], gd[:, 4:4 + k]], axis=1)

    @pl.when(s == 0)
    def _():
        ones = jnp.ones((1, r.shape[0]), F32)
        st[0:1, :rk] += jnp.dot(ones, r, preferred_element_type=F32)
        st[2:2 + rk, :rk] += lax.dot_general(
            r, r, (((0,), (0,)), ((), ())), preferred_element_type=F32)

    @pl.when(jnp.logical_and(s == 1, i == 0))
    def _():
        mr = st[0:1, :rk] * (1.0 / e_total)
        cc = st[2:2 + rk, :rk] * (1.0 / e_total)
        u = _dot(mr, w[0])
        mu1 = u + w[1]
        cw = _dot(cc, w[0])
        m2 = jnp.sum(w[0] * cw, axis=0, keepdims=True)
        var1 = m2 + 2.0 * w[1] * u + w[1] * w[1] - mu1 * mu1
        a1c = lax.rsqrt(var1 + 1e-5) * w[2]
        st[14:15, :] = a1c
        st[15:16, :] = w[3] + (w[1] - mu1) * a1c

    @pl.when(s >= 1)
    def _():
        rw = _dot(r, w[0])
        a1 = jnp.maximum(rw * st[14:15, :] + st[15:16, :], 0.0)
        aw2 = _dot(a1, w[4])

        @pl.when(s == 1)
        def _():
            ones = jnp.ones((1, aw2.shape[0]), F32)
            st[12:13, :] += jnp.dot(ones, aw2, preferred_element_type=F32)
            st[13:14, :] += jnp.dot(ones, aw2 * aw2,
                                    preferred_element_type=F32)

        @pl.when(jnp.logical_and(s == 2, i == 0))
        def _():
            mu_aw = st[12:13, :] * (1.0 / e_total)
            var2 = st[13:14, :] * (1.0 / e_total) - mu_aw * mu_aw
            a2c = lax.rsqrt(var2 + 1e-5) * w[6]
            st[16:17, :] = a2c
            st[17:18, :] = w[7] - mu_aw * a2c

        @pl.when(s == 2)
        def _():
            a2 = jnp.maximum(aw2 * st[16:17, :] + st[17:18, :], 0.0)
            eattr = jnp.maximum(_dot(a2, w[8]) + w[9], 0.0)
            ev = _dot(eattr, we)
            lo = gs[:, 3] + gd[:, 3] + jnp.sum(ev * ae, axis=1)
            lo = jnp.where(lo >= 0, lo, 0.2 * lo)
            wt = jnp.exp(lo)[:, None]
            msg_ref[...] = jnp.concatenate([wt, wt * gs[:, 4:11]], axis=1)


def _edge_stage(gs, gd, mlp_p, we, ae, k, dh):
    e = gs.shape[0]
    nb = e // _BSE
    arrs = [gs, gd] + _mlp_arrs(mlp_p) + [we, ae[None, :]]
    in_specs = ([pl.BlockSpec((_BSE, gs.shape[1]), lambda s, i: (i, 0)),
                 pl.BlockSpec((_BSE, gd.shape[1]), lambda s, i: (i, 0))]
                + _full_specs(arrs[2:]))
    return pl.pallas_call(
        functools.partial(_edge_body, float(e), k, dh),
        grid=(3, nb),
        in_specs=in_specs,
        out_specs=pl.BlockSpec((_BSE, 8), lambda s, i: (i, 0)),
        out_shape=jax.ShapeDtypeStruct((e, 8), F32),
        scratch_shapes=[pltpu.VMEM((18, dh), F32)],
        compiler_params=pltpu.CompilerParams(
            dimension_semantics=("arbitrary", "arbitrary")),
    )(*arrs)


_NW = 32


def _sc_mesh():
    return plsc.VectorSubcoreMesh(core_axis_name="c", subcore_axis_name="s")


def _sc_gather(tsrc, tdst, src, dst):
    n, d = tsrc.shape
    e = src.shape[0]
    ew = e // _NW
    cg = 5000
    nchunk = ew // cg

    @functools.partial(
        pl.kernel,
        mesh=_sc_mesh(),
        out_type=[jax.ShapeDtypeStruct((e, d), F32),
                  jax.ShapeDtypeStruct((e, d), F32)],
        scratch_types=[pltpu.VMEM((cg,), jnp.int32),
                       pltpu.VMEM((cg, d), F32),
                       pltpu.SemaphoreType.DMA],
        compiler_params=pltpu.CompilerParams(use_tc_tiling_on_sc=False),
    )
    def gk(ts_h, td_h, src_h, dst_h, gs_h, gd_h, idx_v, rows_v, sem):
        wid = lax.axis_index("s") * 2 + lax.axis_index("c")
        base_w = wid * ew
        for c in range(nchunk):
            base = base_w + c * cg
            pltpu.sync_copy(src_h.at[pl.ds(base, cg)], idx_v)
            pltpu.async_copy(ts_h.at[idx_v], rows_v, sem).wait()
            pltpu.sync_copy(rows_v, gs_h.at[pl.ds(base, cg)])
            pltpu.sync_copy(dst_h.at[pl.ds(base, cg)], idx_v)
            pltpu.async_copy(td_h.at[idx_v], rows_v, sem).wait()
            pltpu.sync_copy(rows_v, gd_h.at[pl.ds(base, cg)])

    return gk(tsrc, tdst, src, dst)


def _sc_scatter(msg, dst2d, zrs):
    e = msg.shape[0]
    n = zrs.shape[0]
    ew = e // _NW
    cg = 5000
    nchunk = ew // cg
    jrows = cg // 125
    npt = n // 16

    @functools.partial(
        pl.kernel,
        mesh=_sc_mesh(),
        out_type=jax.ShapeDtypeStruct((2 * n, 8), F32),
        scratch_types=[pltpu.VMEM_SHARED((n, 8), F32),
                       pltpu.VMEM((npt, 8), F32),
                       pltpu.VMEM((cg, 8), F32),
                       pltpu.VMEM((jrows, 125), jnp.int32)],
        compiler_params=pltpu.CompilerParams(use_tc_tiling_on_sc=False),
    )
    def sk(msg_h, dst_h, zrs_h, out_h, acc, zb, mb, ib):
        cid = lax.axis_index("c")
        sid = lax.axis_index("s")
        wid = sid * 2 + cid
        pltpu.sync_copy(zrs_h.at[pl.ds(sid * npt, npt)], zb)
        pltpu.sync_copy(zb, acc.at[pl.ds(sid * npt, npt)])
        plsc.subcore_barrier()
        for c in range(nchunk):
            base = wid * ew + c * cg
            pltpu.sync_copy(msg_h.at[pl.ds(base, cg)], mb)
            pltpu.sync_copy(dst_h.at[pl.ds(base // 125, jrows)], ib)

            def scat(j, carry):
                pltpu.sync_copy(mb.at[pl.ds(j * 125, 125)],
                                acc.at[ib.at[j]], add=True)
                return carry

            lax.fori_loop(0, jrows, scat, 0)
        plsc.subcore_barrier()
        pltpu.sync_copy(acc.at[pl.ds(sid * npt, npt)], zb)
        pltpu.sync_copy(zb, out_h.at[pl.ds(cid * n + sid * npt, npt)])

    return sk(msg, dst2d, zrs)


def kernel(x, edge_index, params):
    n = x.shape[0]
    e = edge_index.shape[1]
    p = params
    src = edge_index[0]
    dst = edge_index[1]
    dst2d = dst.reshape(e // 125, 125)
    zrs = jnp.zeros((n, 8), F32)
    g1 = p["conv1_1"]
    g2 = p["conv1_2"]

    src1, dst1, d2 = _node1(x, p, n)

    gs1, gd1 = _sc_gather(src1, dst1, src, dst)
    msg1 = _edge_stage(gs1, gd1, p["mlp1_4"], g1["We"], g1["ae"], 1, 64)
    accp1 = _sc_scatter(msg1, dst2d, zrs)

    src2, dst2 = _node2(accp1, x, d2, p, n)

    gs2, gd2 = _sc_gather(src2, dst2, src, dst)
    msg2 = _edge_stage(gs2, gd2, p["mlp1_8"], g2["We"], g2["ae"], 7, 32)
    accp2 = _sc_scatter(msg2, dst2d, zrs)

    return _node3(accp2, p, n)
